# baseline (speedup 1.0000x reference)
import jax
import jax.numpy as jnp
from jax import lax
from jax.experimental import pallas as pl
from jax.experimental.pallas import tpu as pltpu

_SMOOTH = 0.5
_BETA = 0.1
_MARGIN = 1.0
_SCALE = 0.1
_LANES = 128


def _sep_loss(p):
    k = p.shape[0]
    pn2 = jnp.sum(p * p, axis=1, keepdims=True)
    norm = jnp.sqrt(pn2 + 1e-15)
    maxn = 1.0 - 1e-3
    q = p * jnp.where(norm > maxn, maxn / norm, 1.0)
    g = lax.dot_general(q, q, (((1,), (1,)), ((), ())),
                        preferred_element_type=jnp.float32)
    rows = lax.broadcasted_iota(jnp.int32, (k, k), 0)
    cols = lax.broadcasted_iota(jnp.int32, (k, k), 1)
    eye = (rows == cols).astype(jnp.float32)
    x2 = jnp.sum(g * eye, axis=1, keepdims=True)
    y2 = jnp.sum(g * eye, axis=0, keepdims=True)
    a = 1.0 - 2.0 * g + y2
    b = 1.0 - x2
    num2 = jnp.maximum(a * a * x2 + b * b * y2 - 2.0 * a * b * g, 0.0)
    den = jnp.maximum(1.0 - 2.0 * g + x2 * y2, 1e-15)
    ma2 = num2 / (den * den)
    arg = jnp.clip(jnp.sqrt(ma2 + 1e-15), 0.0, 1.0 - 1e-5)
    dist = jnp.log((1.0 + arg) / (1.0 - arg))
    viol = jnp.maximum(_MARGIN - dist, 0.0) * (1.0 - eye)
    return jnp.sum(viol) / (k * (k - 1))


def _lane_fold(x):
    r = x.shape[-1]
    part = x[..., 0:_LANES]
    for j in range(1, r // _LANES):
        part = part + x[..., j * _LANES:(j + 1) * _LANES]
    return part


def _body(scores_ref, labels_ref, norms_ref, protos_ref, out_ref,
          cnt_acc, nll_acc, reg_acc):
    i = pl.program_id(0)
    nb = pl.num_programs(0)
    r, k = scores_ref.shape

    @pl.when(i == 0)
    def _init():
        cnt_acc[...] = jnp.zeros_like(cnt_acc)
        nll_acc[...] = jnp.zeros_like(nll_acc)
        reg_acc[...] = jnp.zeros_like(reg_acc)

    s = jnp.transpose(scores_ref[...]) * _SCALE
    rs = jnp.sum(jnp.exp(s), axis=0, keepdims=True)
    lse = jnp.log(rs)
    lbl = labels_ref[0]
    oh = lax.broadcasted_iota(jnp.int32, s.shape, 0) == lbl
    cnt_acc[...] += _lane_fold(oh.astype(jnp.float32))
    nll_acc[...] += _lane_fold(jnp.where(oh, lse - s, 0.0))
    nrm = norms_ref[0]
    reg_acc[...] += _lane_fold(nrm * nrm)

    @pl.when(i == nb - 1)
    def _fini():
        raw_cnt = jnp.sum(cnt_acc[...], axis=1, keepdims=True)
        nllsum = jnp.sum(nll_acc[...], axis=1, keepdims=True)
        cnt = jnp.maximum(raw_cnt, 1.0)
        w = jnp.sqrt(jnp.max(cnt) / cnt)
        w = w / (jnp.sum(w) / k)
        ce = jnp.sum(w * nllsum) / jnp.sum(w * raw_cnt)
        reg = jnp.sum(reg_acc[...]) / (nb * r)
        total_loss = ce + _BETA * reg + _sep_loss(protos_ref[...])
        out_ref[...] = jnp.reshape(total_loss, (1, 1))


def kernel(embeddings, scores, labels, prototypes, pre_expmap_norms):
    del embeddings
    m, k = scores.shape
    r = 8192
    nb = m // r
    lbl3 = labels.astype(jnp.int32).reshape(nb, 1, r)
    nrm3 = pre_expmap_norms.reshape(nb, 1, r)
    out = pl.pallas_call(
        _body,
        grid=(nb,),
        in_specs=[
            pl.BlockSpec((r, k), lambda i: (i, 0)),
            pl.BlockSpec((1, 1, r), lambda i: (i, 0, 0)),
            pl.BlockSpec((1, 1, r), lambda i: (i, 0, 0)),
            pl.BlockSpec(prototypes.shape, lambda i: (0, 0)),
        ],
        out_specs=pl.BlockSpec((1, 1), lambda i: (0, 0)),
        out_shape=jax.ShapeDtypeStruct((1, 1), jnp.float32),
        scratch_shapes=[
            pltpu.VMEM((k, _LANES), jnp.float32),
            pltpu.VMEM((k, _LANES), jnp.float32),
            pltpu.VMEM((1, _LANES), jnp.float32),
        ],
        compiler_params=pltpu.CompilerParams(
            dimension_semantics=("arbitrary",)),
    )(scores, lbl3, nrm3, prototypes)
    return out[0, 0]


# speedup vs baseline: 3.0481x; 3.0481x over previous
import jax
import jax.numpy as jnp
from jax import lax
from jax.experimental import pallas as pl
from jax.experimental.pallas import tpu as pltpu

_SMOOTH = 0.5
_BETA = 0.1
_MARGIN = 1.0
_SCALE = 0.1
_LANES = 128


def _sep_loss(p):
    k = p.shape[0]
    pn2 = jnp.sum(p * p, axis=1, keepdims=True)
    norm = jnp.sqrt(pn2 + 1e-15)
    maxn = 1.0 - 1e-3
    q = p * jnp.where(norm > maxn, maxn / norm, 1.0)
    g = lax.dot_general(q, q, (((1,), (1,)), ((), ())),
                        preferred_element_type=jnp.float32)
    rows = lax.broadcasted_iota(jnp.int32, (k, k), 0)
    cols = lax.broadcasted_iota(jnp.int32, (k, k), 1)
    eye = (rows == cols).astype(jnp.float32)
    x2 = jnp.sum(g * eye, axis=1, keepdims=True)
    y2 = jnp.sum(g * eye, axis=0, keepdims=True)
    a = 1.0 - 2.0 * g + y2
    b = 1.0 - x2
    num2 = jnp.maximum(a * a * x2 + b * b * y2 - 2.0 * a * b * g, 0.0)
    den = jnp.maximum(1.0 - 2.0 * g + x2 * y2, 1e-15)
    ma2 = num2 / (den * den)
    arg = jnp.clip(jnp.sqrt(ma2 + 1e-15), 0.0, 1.0 - 1e-5)
    dist = jnp.log((1.0 + arg) / (1.0 - arg))
    viol = jnp.maximum(_MARGIN - dist, 0.0) * (1.0 - eye)
    return jnp.sum(viol) / (k * (k - 1))


def _lane_fold(x):
    r = x.shape[-1]
    part = x[..., 0:_LANES]
    for j in range(1, r // _LANES):
        part = part + x[..., j * _LANES:(j + 1) * _LANES]
    return part


def _body(scores_ref, labels_ref, norms_ref, protos_ref, out_ref,
          cnt_acc, nll_acc, reg_acc):
    i = pl.program_id(0)
    nb = pl.num_programs(0)
    k, r = scores_ref.shape

    @pl.when(i == 0)
    def _init():
        cnt_acc[...] = jnp.zeros_like(cnt_acc)
        nll_acc[...] = jnp.zeros_like(nll_acc)
        reg_acc[...] = jnp.zeros_like(reg_acc)

    s = scores_ref[...] * _SCALE
    rs = jnp.sum(jnp.exp(s), axis=0, keepdims=True)
    lse = jnp.log(rs)
    lbl = labels_ref[0]
    oh = lax.broadcasted_iota(jnp.int32, s.shape, 0) == lbl
    cnt_acc[...] += _lane_fold(oh.astype(jnp.float32))
    nll_acc[...] += _lane_fold(jnp.where(oh, lse - s, 0.0))
    nrm = norms_ref[0]
    reg_acc[...] += _lane_fold(nrm * nrm)

    @pl.when(i == nb - 1)
    def _fini():
        raw_cnt = jnp.sum(cnt_acc[...], axis=1, keepdims=True)
        nllsum = jnp.sum(nll_acc[...], axis=1, keepdims=True)
        cnt = jnp.maximum(raw_cnt, 1.0)
        w = jnp.sqrt(jnp.max(cnt) / cnt)
        w = w / (jnp.sum(w) / k)
        ce = jnp.sum(w * nllsum) / jnp.sum(w * raw_cnt)
        reg = jnp.sum(reg_acc[...]) / (nb * r)
        p = jnp.transpose(protos_ref[...])
        total_loss = ce + _BETA * reg + _sep_loss(p)
        out_ref[...] = jnp.reshape(total_loss, (1, 1))


def kernel(embeddings, scores, labels, prototypes, pre_expmap_norms):
    del embeddings
    m, k = scores.shape
    d = prototypes.shape[1]
    r = 8192
    nb = m // r
    scores_t = scores.T
    protos_t = prototypes.T
    lbl3 = labels.astype(jnp.int32).reshape(nb, 1, r)
    nrm3 = pre_expmap_norms.reshape(nb, 1, r)
    out = pl.pallas_call(
        _body,
        grid=(nb,),
        in_specs=[
            pl.BlockSpec((k, r), lambda i: (0, i)),
            pl.BlockSpec((1, 1, r), lambda i: (i, 0, 0)),
            pl.BlockSpec((1, 1, r), lambda i: (i, 0, 0)),
            pl.BlockSpec((d, k), lambda i: (0, 0)),
        ],
        out_specs=pl.BlockSpec((1, 1), lambda i: (0, 0)),
        out_shape=jax.ShapeDtypeStruct((1, 1), jnp.float32),
        scratch_shapes=[
            pltpu.VMEM((k, _LANES), jnp.float32),
            pltpu.VMEM((k, _LANES), jnp.float32),
            pltpu.VMEM((1, _LANES), jnp.float32),
        ],
        compiler_params=pltpu.CompilerParams(
            dimension_semantics=("arbitrary",)),
    )(scores_t, lbl3, nrm3, protos_t)
    return out[0, 0]


# speedup vs baseline: 3.1630x; 1.0377x over previous
import jax
import jax.numpy as jnp
from jax import lax
from jax.experimental import pallas as pl
from jax.experimental.pallas import tpu as pltpu

_SMOOTH = 0.5
_BETA = 0.1
_MARGIN = 1.0
_SCALE = 0.1
_LANES = 128


def _sep_loss(p):
    k = p.shape[0]
    pn2 = jnp.sum(p * p, axis=1, keepdims=True)
    norm = jnp.sqrt(pn2 + 1e-15)
    maxn = 1.0 - 1e-3
    q = p * jnp.where(norm > maxn, maxn / norm, 1.0)
    g = lax.dot_general(q, q, (((1,), (1,)), ((), ())),
                        preferred_element_type=jnp.float32)
    rows = lax.broadcasted_iota(jnp.int32, (k, k), 0)
    cols = lax.broadcasted_iota(jnp.int32, (k, k), 1)
    eye = (rows == cols).astype(jnp.float32)
    x2 = jnp.sum(g * eye, axis=1, keepdims=True)
    y2 = jnp.sum(g * eye, axis=0, keepdims=True)
    a = 1.0 - 2.0 * g + y2
    b = 1.0 - x2
    num2 = jnp.maximum(a * a * x2 + b * b * y2 - 2.0 * a * b * g, 0.0)
    den = jnp.maximum(1.0 - 2.0 * g + x2 * y2, 1e-15)
    ma2 = num2 / (den * den)
    arg = jnp.clip(jnp.sqrt(ma2 + 1e-15), 0.0, 1.0 - 1e-5)
    dist = jnp.log((1.0 + arg) / (1.0 - arg))
    viol = jnp.maximum(_MARGIN - dist, 0.0) * (1.0 - eye)
    return jnp.sum(viol) / (k * (k - 1))


def _lane_fold(x):
    r = x.shape[-1]
    part = x[..., 0:_LANES]
    for j in range(1, r // _LANES):
        part = part + x[..., j * _LANES:(j + 1) * _LANES]
    return part


def _body(scores_ref, labels_ref, norms_ref, protos_ref, out_ref,
          cnt_acc, nll_acc, reg_acc):
    i = pl.program_id(0)
    nb = pl.num_programs(0)
    k, r = scores_ref.shape

    @pl.when(i == 0)
    def _init():
        cnt_acc[...] = jnp.zeros_like(cnt_acc)
        nll_acc[...] = jnp.zeros_like(nll_acc)
        reg_acc[...] = jnp.zeros_like(reg_acc)

    s = scores_ref[...] * _SCALE
    rs = jnp.sum(jnp.exp(s), axis=0, keepdims=True)
    lse = jnp.log(rs)
    lbl = labels_ref[0]
    oh = lax.broadcasted_iota(jnp.int32, s.shape, 0) == lbl
    cnt_acc[...] += _lane_fold(oh.astype(jnp.float32))
    nll_acc[...] += _lane_fold(jnp.where(oh, lse - s, 0.0))
    nrm = norms_ref[0]
    reg_acc[...] += _lane_fold(nrm * nrm)

    @pl.when(i == nb - 1)
    def _fini():
        raw_cnt = jnp.sum(cnt_acc[...], axis=1, keepdims=True)
        nllsum = jnp.sum(nll_acc[...], axis=1, keepdims=True)
        cnt = jnp.maximum(raw_cnt, 1.0)
        w = jnp.sqrt(jnp.max(cnt) / cnt)
        w = w / (jnp.sum(w) / k)
        ce = jnp.sum(w * nllsum) / jnp.sum(w * raw_cnt)
        reg = jnp.sum(reg_acc[...]) / (nb * r)
        p = jnp.transpose(protos_ref[...])
        total_loss = ce + _BETA * reg + _sep_loss(p)
        out_ref[...] = jnp.reshape(total_loss, (1, 1))


def kernel(embeddings, scores, labels, prototypes, pre_expmap_norms):
    del embeddings
    m, k = scores.shape
    d = prototypes.shape[1]
    r = 16384
    nb = m // r
    scores_t = scores.T
    protos_t = prototypes.T
    lbl3 = labels.astype(jnp.int32).reshape(nb, 1, r)
    nrm3 = pre_expmap_norms.reshape(nb, 1, r)
    out = pl.pallas_call(
        _body,
        grid=(nb,),
        in_specs=[
            pl.BlockSpec((k, r), lambda i: (0, i)),
            pl.BlockSpec((1, 1, r), lambda i: (i, 0, 0)),
            pl.BlockSpec((1, 1, r), lambda i: (i, 0, 0)),
            pl.BlockSpec((d, k), lambda i: (0, 0)),
        ],
        out_specs=pl.BlockSpec((1, 1), lambda i: (0, 0)),
        out_shape=jax.ShapeDtypeStruct((1, 1), jnp.float32),
        scratch_shapes=[
            pltpu.VMEM((k, _LANES), jnp.float32),
            pltpu.VMEM((k, _LANES), jnp.float32),
            pltpu.VMEM((1, _LANES), jnp.float32),
        ],
        compiler_params=pltpu.CompilerParams(
            dimension_semantics=("arbitrary",)),
    )(scores_t, lbl3, nrm3, protos_t)
    return out[0, 0]
